# baseline (speedup 1.0000x reference)
import functools

import jax
import jax.numpy as jnp
from jax import lax
from jax.experimental import pallas as pl
from jax.experimental.pallas import tpu as pltpu
from jax.experimental.pallas import tpu_sc as plsc

B = 32
N = 16384
S = 64
L = 16
CP0 = 4096
CP1 = N - CP0
UNROLL = 4
RADIUS_SQ = 1.0
FULL = 0x40404040

_mesh = plsc.VectorSubcoreMesh(core_axis_name="c", subcore_axis_name="s")


@functools.partial(
    pl.kernel,
    mesh=_mesh,
    compiler_params=pltpu.CompilerParams(needs_layout_passes=False),
    out_type=jax.ShapeDtypeStruct((B, 8, S), jnp.int32),
    scratch_types=[
        pltpu.VMEM((3, N), jnp.float32),
        pltpu.VMEM((8, S), jnp.int32),
        pltpu.SemaphoreType.DMA((2,)),
    ],
)
def _octant_query_sc(pcs_hbm, out_hbm, pts, obuf, dsems):
    wid = lax.axis_index("s") * 2 + lax.axis_index("c")

    copies = [
        pltpu.async_copy(pcs_hbm.at[:, wid, pl.ds(0, CP0)],
                         pts.at[:, pl.ds(0, CP0)], dsems.at[0]),
        pltpu.async_copy(pcs_hbm.at[:, wid, pl.ds(CP0, CP1)],
                         pts.at[:, pl.ds(CP0, CP1)], dsems.at[1]),
    ]

    lane = lax.iota(jnp.int32, L)
    neg1 = jnp.full((L,), -1, jnp.int32)
    for o in range(8):
        for j in range(S // L):
            obuf[o, pl.ds(j * L, L)] = neg1
    last = jnp.full((L,), L - 1, jnp.int32)

    def do_vreg(j, cl, ch):
        off = j * L
        x = pts[0, pl.ds(off, L)]
        y = pts[1, pl.ds(off, L)]
        z = pts[2, pl.ds(off, L)]
        oct_id = ((x > 0).astype(jnp.int32) * 4
                  + (y > 0).astype(jnp.int32) * 2
                  + (z > 0).astype(jnp.int32))
        within = (x * x + y * y + z * z) <= RADIUS_SQ
        low = oct_id < 4
        shamt = (oct_id & 3) << 3
        oh = jnp.left_shift(jnp.int32(1), shamt)
        zero = jnp.zeros((L,), jnp.int32)
        ohl = jnp.where(within & low, oh, zero)
        ohh = jnp.where(within & jnp.logical_not(low), oh, zero)
        cuml = plsc.cumsum(ohl)
        cumh = plsc.cumsum(ohh)
        incl = (jnp.where(low, cuml, cumh) >> shamt) & 255
        cnt = (jnp.where(low, cl, ch) >> shamt) & 255
        slot = cnt + incl - 1
        sel = within & (slot < S)
        slot_c = jnp.where(sel, slot, 0)
        plsc.store_scatter(obuf, [oct_id, slot_c], lane + off, mask=sel)
        tot_l = cuml.at[last].get(mode="promise_in_bounds")
        tot_h = cumh.at[last].get(mode="promise_in_bounds")
        return cl + tot_l, ch + tot_h

    def clamp64(c):
        m = ((c >> 6) | (c >> 7)) & 0x01010101
        fm = (m << 8) - m
        return (c & ~fm) | (m << 6)

    done = jnp.bool_(False)
    cl = jnp.zeros((L,), jnp.int32)
    ch = jnp.zeros((L,), jnp.int32)
    bounds = (CP0 // L, N // L)
    start = jnp.int32(0)
    for c in range(2):
        copies[c].wait()

        def cond(carry):
            j, _, _, d = carry
            return (j < bounds[c]) & jnp.logical_not(d)

        def body(carry):
            j, cl, ch, _ = carry
            for u in range(UNROLL):
                cl, ch = do_vreg(j + u, cl, ch)
            cl = clamp64(cl)
            ch = clamp64(ch)
            d = jnp.all((cl == FULL) & (ch == FULL))
            return j + UNROLL, cl, ch, d

        start, cl, ch, done = lax.while_loop(cond, body,
                                             (start, cl, ch, done))

    pltpu.sync_copy(obuf, out_hbm.at[wid])


def kernel(pcs):
    return _octant_query_sc(jnp.transpose(pcs, (1, 0, 2)))


# speedup vs baseline: 1.0091x; 1.0091x over previous
import functools

import jax
import jax.numpy as jnp
from jax import lax
from jax.experimental import pallas as pl
from jax.experimental.pallas import tpu as pltpu
from jax.experimental.pallas import tpu_sc as plsc

B = 32
N = 16384
S = 64
L = 16
CP0 = 2048
CP1 = N - CP0
UNROLL = 4
RADIUS_SQ = 1.0
FULL = 0x40404040

_mesh = plsc.VectorSubcoreMesh(core_axis_name="c", subcore_axis_name="s")


@functools.partial(
    pl.kernel,
    mesh=_mesh,
    compiler_params=pltpu.CompilerParams(needs_layout_passes=False),
    out_type=jax.ShapeDtypeStruct((B, 8, S), jnp.int32),
    scratch_types=[
        pltpu.VMEM((3, N), jnp.float32),
        pltpu.VMEM((8, S), jnp.int32),
        pltpu.SemaphoreType.DMA((2,)),
    ],
)
def _octant_query_sc(pcs_hbm, out_hbm, pts, obuf, dsems):
    wid = lax.axis_index("s") * 2 + lax.axis_index("c")

    copies = [
        pltpu.async_copy(pcs_hbm.at[:, wid, pl.ds(0, CP0)],
                         pts.at[:, pl.ds(0, CP0)], dsems.at[0]),
        pltpu.async_copy(pcs_hbm.at[:, wid, pl.ds(CP0, CP1)],
                         pts.at[:, pl.ds(CP0, CP1)], dsems.at[1]),
    ]

    lane = lax.iota(jnp.int32, L)
    neg1 = jnp.full((L,), -1, jnp.int32)
    for o in range(8):
        for j in range(S // L):
            obuf[o, pl.ds(j * L, L)] = neg1
    last = jnp.full((L,), L - 1, jnp.int32)

    def do_vreg(j, cl, ch):
        off = j * L
        x = pts[0, pl.ds(off, L)]
        y = pts[1, pl.ds(off, L)]
        z = pts[2, pl.ds(off, L)]
        oct_id = ((x > 0).astype(jnp.int32) * 4
                  + (y > 0).astype(jnp.int32) * 2
                  + (z > 0).astype(jnp.int32))
        within = (x * x + y * y + z * z) <= RADIUS_SQ
        low = oct_id < 4
        shamt = (oct_id & 3) << 3
        oh = jnp.left_shift(jnp.int32(1), shamt)
        zero = jnp.zeros((L,), jnp.int32)
        ohl = jnp.where(within & low, oh, zero)
        ohh = jnp.where(within & jnp.logical_not(low), oh, zero)
        cuml = plsc.cumsum(ohl)
        cumh = plsc.cumsum(ohh)
        incl = (jnp.where(low, cuml, cumh) >> shamt) & 255
        cnt = (jnp.where(low, cl, ch) >> shamt) & 255
        slot = cnt + incl - 1
        sel = within & (slot < S)
        slot_c = jnp.where(sel, slot, 0)
        plsc.store_scatter(obuf, [oct_id, slot_c], lane + off, mask=sel)
        tot_l = cuml.at[last].get(mode="promise_in_bounds")
        tot_h = cumh.at[last].get(mode="promise_in_bounds")
        return cl + tot_l, ch + tot_h

    def clamp64(c):
        m = ((c >> 6) | (c >> 7)) & 0x01010101
        fm = (m << 8) - m
        return (c & ~fm) | (m << 6)

    done = jnp.bool_(False)
    cl = jnp.zeros((L,), jnp.int32)
    ch = jnp.zeros((L,), jnp.int32)
    bounds = (CP0 // L, N // L)
    start = jnp.int32(0)
    for c in range(2):
        copies[c].wait()

        def cond(carry):
            j, _, _, d = carry
            return (j < bounds[c]) & jnp.logical_not(d)

        def body(carry):
            j, cl, ch, _ = carry
            for u in range(UNROLL):
                cl, ch = do_vreg(j + u, cl, ch)
            cl = clamp64(cl)
            ch = clamp64(ch)
            d = jnp.all((cl == FULL) & (ch == FULL))
            return j + UNROLL, cl, ch, d

        start, cl, ch, done = lax.while_loop(cond, body,
                                             (start, cl, ch, done))

    pltpu.sync_copy(obuf, out_hbm.at[wid])


def kernel(pcs):
    return _octant_query_sc(jnp.transpose(pcs, (1, 0, 2)))
